# baseline (speedup 1.0000x reference)
import functools

import jax
import jax.numpy as jnp
from jax import lax
from jax.experimental import pallas as pl
from jax.experimental.pallas import tpu as pltpu
from jax.experimental.pallas import tpu_sc as plsc

B, N, D = 256, 1024, 96
K = 512
R = B * K

NC, NS = 2, 16
NW = NC * NS
ROWS_PER_W = R // NW
CHUNK = 128
NCHUNK = ROWS_PER_W // CHUNK


def _keep_indices():
    scores = jax.random.normal(jax.random.key(42), (B, N), dtype=jnp.float32)
    _, keep = jax.lax.top_k(scores, K)
    flat = keep.astype(jnp.int32) + (jnp.arange(B, dtype=jnp.int32) * N)[:, None]
    return flat.reshape(NW, NCHUNK, CHUNK)


_mesh = plsc.VectorSubcoreMesh(core_axis_name="c", subcore_axis_name="s")


@functools.partial(
    pl.kernel,
    mesh=_mesh,
    out_type=jax.ShapeDtypeStruct((R, D), jnp.float32),
    compiler_params=pltpu.CompilerParams(use_tc_tiling_on_sc=False),
    scratch_types=[
        pltpu.VMEM((NCHUNK, CHUNK), jnp.int32),
        pltpu.VMEM((CHUNK, D), jnp.float32),
        pltpu.SemaphoreType.DMA,
    ],
)
def _gather_rows(table_hbm, idx_hbm, out_hbm, idx_v, buf_v, gsem):
    wid = lax.axis_index("s") * NC + lax.axis_index("c")
    base = wid * ROWS_PER_W
    pltpu.sync_copy(idx_hbm.at[wid], idx_v)

    def body(j, carry):
        pltpu.async_copy(table_hbm.at[idx_v.at[j]], buf_v, gsem).wait()
        pltpu.sync_copy(buf_v, out_hbm.at[pl.ds(base + j * CHUNK, CHUNK)])
        return carry

    lax.fori_loop(0, NCHUNK, body, 0)


def kernel(x):
    idx = _keep_indices()
    table = x.reshape(B * N, D)
    out = _gather_rows(table, idx)
    return out.reshape(B, K, D)


# speedup vs baseline: 1.4528x; 1.4528x over previous
import functools

import jax
import jax.numpy as jnp
import numpy as np
from jax import lax
from jax.experimental import pallas as pl
from jax.experimental.pallas import tpu as pltpu
from jax.experimental.pallas import tpu_sc as plsc

B, N, D = 256, 1024, 96
K = 512
R = B * K

NC, NS = 2, 16
NW = NC * NS
BPW = B // NW
CHUNK = 128
NCHUNK = N // CHUNK
PB = K + NCHUNK * 15
PB = ((PB + 15) // 16) * 16
TRASH = K

_mesh = plsc.VectorSubcoreMesh(core_axis_name="c", subcore_axis_name="s")


@functools.cache
def _plan():
    with jax.ensure_compile_time_eval():
        scores = jax.random.normal(jax.random.key(42), (B, N), dtype=jnp.float32)
        _, keep = jax.lax.top_k(scores, K)
        keep = np.asarray(keep)
    src = keep.astype(np.int64)
    chunk = src >> 7
    order = np.argsort(chunk, axis=1, kind="stable")
    src_s = np.take_along_axis(src, order, axis=1)
    dst_s = order
    chunk_s = src_s >> 7
    cnt = (chunk_s[:, :, None] == np.arange(NCHUNK)).sum(axis=1)
    pcnt = ((cnt + 15) // 16) * 16
    zeros = np.zeros((B, 1), dtype=np.int64)
    poff = np.concatenate([zeros, np.cumsum(pcnt, axis=1)[:, :-1]], axis=1)
    startc = np.concatenate([zeros, np.cumsum(cnt, axis=1)[:, :-1]], axis=1)
    rank = np.arange(K)[None, :] - np.take_along_axis(startc, chunk_s, axis=1)
    pos = np.take_along_axis(poff, chunk_s, axis=1) + rank
    pairs = np.full((B, PB), TRASH << 16, dtype=np.int32)
    word = (src_s & 127) | (dst_s << 16)
    pairs[np.arange(B)[:, None], pos] = word.astype(np.int32)
    meta = np.concatenate([poff >> 4, pcnt >> 4], axis=1).astype(np.int32)
    return pairs.reshape(NW, BPW * PB), meta.reshape(NW, BPW * 16)


_SCRATCH = [
    pltpu.VMEM((BPW * PB,), jnp.int32),
    pltpu.VMEM((BPW * 16,), jnp.int32),
    pltpu.VMEM((CHUNK, D), jnp.float32),
    pltpu.VMEM((CHUNK, D), jnp.float32),
    pltpu.VMEM((K + 1, D), jnp.float32),
    pltpu.SemaphoreType.DMA,
    pltpu.SemaphoreType.DMA,
    pltpu.SemaphoreType.DMA,
]


def _body(
    x_hbm, pairs_hbm, meta_hbm, out_hbm,
    pairs_v, meta_v, buf0, buf1, obuf, sem0, sem1, semo,
):
    wid = lax.axis_index("s") * NC + lax.axis_index("c")
    pltpu.sync_copy(pairs_hbm.at[wid], pairs_v)
    pltpu.sync_copy(meta_hbm.at[wid], meta_v)
    bufs = (buf0, buf1)
    sems = (sem0, sem1)

    def batch_body(bl, carry):
        gx = (wid * BPW + bl) * N
        gout = (wid * BPW + bl) * K
        h0 = pltpu.async_copy(x_hbm.at[pl.ds(gx, CHUNK)], buf0, sem0)

        @pl.when(bl > 0)
        def _():
            pltpu.make_async_copy(
                out_hbm.at[pl.ds(0, K)], obuf.at[pl.ds(0, K)], semo
            ).wait()

        mv = meta_v[pl.ds(bl * 16, 16)]
        handles = [h0] + [None] * (NCHUNK - 1)
        for c in range(NCHUNK):
            if c + 1 < NCHUNK:
                handles[c + 1] = pltpu.async_copy(
                    x_hbm.at[pl.ds(gx + (c + 1) * CHUNK, CHUNK)],
                    bufs[(c + 1) % 2],
                    sems[(c + 1) % 2],
                )
            handles[c].wait()
            goff = mv[c]
            gcnt = mv[8 + c]
            buf = bufs[c % 2]

            def group_body(g, inner, buf=buf, goff=goff):
                pv = pairs_v[pl.ds(bl * PB + (goff + g) * 16, 16)]
                srcs = jnp.bitwise_and(pv, 127)
                dsts = lax.shift_right_logical(pv, 16)
                for t in range(16):
                    s = srcs[t]
                    d = dsts[t]
                    for j in range(D // 16):
                        obuf[d, pl.ds(j * 16, 16)] = buf[s, pl.ds(j * 16, 16)]
                return inner

            lax.fori_loop(0, gcnt, group_body, 0)

        pltpu.async_copy(obuf.at[pl.ds(0, K)], out_hbm.at[pl.ds(gout, K)], semo)
        return carry

    lax.fori_loop(0, BPW, batch_body, 0)
    pltpu.make_async_copy(
        out_hbm.at[pl.ds(0, K)], obuf.at[pl.ds(0, K)], semo
    ).wait()


_gather_tiled = pl.kernel(
    _body,
    mesh=_mesh,
    out_type=jax.ShapeDtypeStruct((R, D), jnp.float32),
    compiler_params=pltpu.CompilerParams(use_tc_tiling_on_sc=True),
    scratch_types=_SCRATCH,
)


def kernel(x):
    pairs, meta = _plan()
    out = _gather_tiled(
        x.reshape(B * N, D), jnp.asarray(pairs), jnp.asarray(meta)
    )
    return out.reshape(B, K, D)


# speedup vs baseline: 2.6577x; 1.8294x over previous
import functools

import jax
import jax.numpy as jnp
import numpy as np
from jax import lax
from jax.experimental import pallas as pl
from jax.experimental.pallas import tpu as pltpu
from jax.experimental.pallas import tpu_sc as plsc

B, N, D = 256, 1024, 96
K = 512

NC, NS = 2, 16
NW = NC * NS
BPW = B // NW
DCHUNK = 32
NDC = D // DCHUNK
NG = K // 16

_mesh = plsc.VectorSubcoreMesh(core_axis_name="c", subcore_axis_name="s")


@functools.cache
def _keep_indices():
    with jax.ensure_compile_time_eval():
        scores = jax.random.normal(jax.random.key(42), (B, N), dtype=jnp.float32)
        _, keep = jax.lax.top_k(scores, K)
        return np.asarray(keep).astype(np.int32).reshape(NW, BPW * K)


_SCRATCH = [
    pltpu.VMEM((BPW * K,), jnp.int32),
    pltpu.VMEM((DCHUNK, N), jnp.float32),
    pltpu.VMEM((DCHUNK, N), jnp.float32),
    pltpu.VMEM((D, K), jnp.float32),
    pltpu.SemaphoreType.DMA,
    pltpu.SemaphoreType.DMA,
    pltpu.SemaphoreType.DMA,
]


def _body(xt_hbm, idx_hbm, out_hbm, idx_v, buf0, buf1, obuf, sem0, sem1, semo):
    wid = lax.axis_index("s") * NC + lax.axis_index("c")
    pltpu.sync_copy(idx_hbm.at[wid], idx_v)
    bufs = (buf0, buf1)
    sems = (sem0, sem1)
    dz = jnp.zeros((16,), jnp.int32)

    def batch_body(bl, carry):
        b = wid * BPW + bl
        h0 = pltpu.async_copy(xt_hbm.at[b, pl.ds(0, DCHUNK)], buf0, sem0)

        @pl.when(bl > 0)
        def _():
            pltpu.make_async_copy(out_hbm.at[0], obuf, semo).wait()

        for c in range(NDC):
            if c + 1 < NDC:
                pltpu.async_copy(
                    xt_hbm.at[b, pl.ds((c + 1) * DCHUNK, DCHUNK)],
                    bufs[(c + 1) % 2],
                    sems[(c + 1) % 2],
                )
            if c == 0:
                h0.wait()
            else:
                pltpu.make_async_copy(
                    xt_hbm.at[b, pl.ds(0, DCHUNK)], bufs[c % 2], sems[c % 2]
                ).wait()
            buf = bufs[c % 2]

            def group_body(g, inner, buf=buf, c=c):
                srcs = idx_v[pl.ds(bl * K + g * 16, 16)]
                for d in range(DCHUNK):
                    val = plsc.load_gather(buf, [dz + d, srcs])
                    obuf[c * DCHUNK + d, pl.ds(g * 16, 16)] = val
                return inner

            lax.fori_loop(0, NG, group_body, 0)

        pltpu.async_copy(obuf, out_hbm.at[b], semo)
        return carry

    lax.fori_loop(0, BPW, batch_body, 0)
    pltpu.make_async_copy(out_hbm.at[0], obuf, semo).wait()


_gather_t = pl.kernel(
    _body,
    mesh=_mesh,
    out_type=jax.ShapeDtypeStruct((B, D, K), jnp.float32),
    compiler_params=pltpu.CompilerParams(
        use_tc_tiling_on_sc=True, needs_layout_passes=False
    ),
    scratch_types=_SCRATCH,
)


def kernel(x):
    idx = jnp.asarray(_keep_indices())
    out_t = _gather_t(jnp.swapaxes(x, 1, 2), idx)
    return jnp.swapaxes(out_t, 1, 2)


# speedup vs baseline: 4.1536x; 1.5629x over previous
import functools

import jax
import jax.numpy as jnp
import numpy as np
from jax import lax
from jax.experimental import pallas as pl
from jax.experimental.pallas import tpu as pltpu
from jax.experimental.pallas import tpu_sc as plsc

B, N, D = 256, 1024, 96
K = 512

NC, NS = 2, 16
NW = NC * NS
BPW = B // NW
DCHUNK = 32
NDC = D // DCHUNK
NG = K // 16

_mesh = plsc.VectorSubcoreMesh(core_axis_name="c", subcore_axis_name="s")


@functools.cache
def _keep_indices():
    with jax.ensure_compile_time_eval():
        scores = jax.random.normal(jax.random.key(42), (B, N), dtype=jnp.float32)
        _, keep = jax.lax.top_k(scores, K)
        return np.asarray(keep).astype(np.int32).reshape(NW, BPW * K)


_SCRATCH = [
    pltpu.VMEM((BPW * K,), jnp.int32),
    pltpu.VMEM((DCHUNK, N), jnp.float32),
    pltpu.VMEM((DCHUNK, N), jnp.float32),
    pltpu.VMEM((D, K), jnp.float32),
    pltpu.SemaphoreType.DMA,
    pltpu.SemaphoreType.DMA,
    pltpu.SemaphoreType.DMA,
]


def _body(xt_hbm, idx_hbm, out_hbm, idx_v, buf0, buf1, obuf, sem0, sem1, semo):
    wid = lax.axis_index("s") * NC + lax.axis_index("c")
    pltpu.sync_copy(idx_hbm.at[wid], idx_v)
    bufs = (buf0, buf1)
    sems = (sem0, sem1)
    dz = jnp.zeros((16,), jnp.int32)

    def batch_body(bl, carry):
        b = wid * BPW + bl
        h0 = pltpu.async_copy(xt_hbm.at[b, pl.ds(0, DCHUNK)], buf0, sem0)

        @pl.when(bl > 0)
        def _():
            pltpu.make_async_copy(out_hbm.at[0], obuf, semo).wait()

        for c in range(NDC):
            if c + 1 < NDC:
                pltpu.async_copy(
                    xt_hbm.at[b, pl.ds((c + 1) * DCHUNK, DCHUNK)],
                    bufs[(c + 1) % 2],
                    sems[(c + 1) % 2],
                )
            if c == 0:
                h0.wait()
            else:
                pltpu.make_async_copy(
                    xt_hbm.at[b, pl.ds(0, DCHUNK)], bufs[c % 2], sems[c % 2]
                ).wait()
            buf = bufs[c % 2]

            def group_body(g, inner, buf=buf, c=c):
                srcs = idx_v[pl.ds(bl * K + g * 16, 16)]
                vals = [
                    plsc.load_gather(buf, [dz + d, srcs])
                    for d in range(DCHUNK)
                ]
                for d in range(DCHUNK):
                    obuf[c * DCHUNK + d, pl.ds(g * 16, 16)] = vals[d]
                return inner

            lax.fori_loop(0, NG, group_body, 0)

        pltpu.async_copy(obuf, out_hbm.at[b], semo)
        return carry

    lax.fori_loop(0, BPW, batch_body, 0)
    pltpu.make_async_copy(out_hbm.at[0], obuf, semo).wait()


_gather_t = pl.kernel(
    _body,
    mesh=_mesh,
    out_type=jax.ShapeDtypeStruct((B, D, K), jnp.float32),
    compiler_params=pltpu.CompilerParams(
        use_tc_tiling_on_sc=True, needs_layout_passes=False
    ),
    scratch_types=_SCRATCH,
)


def kernel(x):
    idx = jnp.asarray(_keep_indices())
    out_t = _gather_t(jnp.swapaxes(x, 1, 2), idx)
    return jnp.swapaxes(out_t, 1, 2)
